# Initial kernel scaffold; baseline (speedup 1.0000x reference)
import jax
import jax.numpy as jnp
from jax.experimental import pallas as pl


def kernel(encoding, encoding_pos, object, object_pos, edge_index_object, edge_index_cross, params):
    raise NotImplementedError("write your pallas kernel here")


# speedup vs baseline: 3.4021x; 3.4021x over previous
import jax
import jax.numpy as jnp
from jax import lax
from jax.experimental import pallas as pl
from jax.experimental.pallas import tpu as pltpu
from jax.experimental.pallas import tpu_sc as plsc

C = 256
P = 3
HW = 128
RADIUS = 1.0
NC, NS = 2, 16
_SCALE = 1.0 / 16.0


def _row_grid(n, bm=1000):
    assert n % bm == 0
    return n // bm, bm


def _prep_dst(x, wq, bq):
    n = x.shape[0]
    grid, bm = _row_grid(n)

    def f(x_ref, wq_ref, bq_ref, qx_ref):
        q = jnp.dot(x_ref[...], wq_ref[...], preferred_element_type=jnp.float32)
        qx_ref[...] = (q + bq_ref[...]) * _SCALE

    return pl.pallas_call(
        f,
        grid=(grid,),
        in_specs=[
            pl.BlockSpec((bm, C), lambda i: (i, 0)),
            pl.BlockSpec((C, C), lambda i: (0, 0)),
            pl.BlockSpec((1, C), lambda i: (0, 0)),
        ],
        out_specs=pl.BlockSpec((bm, C), lambda i: (i, 0)),
        out_shape=jax.ShapeDtypeStruct((n, C), jnp.float32),
    )(x, wq, bq.reshape(1, C))


def _prep_src(x, pos, wk, bk, wv, bv, wp):
    n = x.shape[0]
    grid, bm = _row_grid(n)

    def f(x_ref, pos_ref, wk_ref, bk_ref, wv_ref, bv_ref, wp_ref,
          kx_ref, ua_ref, ub_ref):
        xv = x_ref[...]
        pp = jnp.dot(pos_ref[...], wp_ref[...],
                     preferred_element_type=jnp.float32)
        k = jnp.dot(xv, wk_ref[...], preferred_element_type=jnp.float32)
        v = jnp.dot(xv, wv_ref[...], preferred_element_type=jnp.float32)
        kx_ref[...] = k + bk_ref[...] + pp
        u = v + bv_ref[...] + pp
        ua_ref[...] = u[:, :HW]
        ub_ref[...] = u[:, HW:]

    return pl.pallas_call(
        f,
        grid=(grid,),
        in_specs=[
            pl.BlockSpec((bm, C), lambda i: (i, 0)),
            pl.BlockSpec((bm, P), lambda i: (i, 0)),
            pl.BlockSpec((C, C), lambda i: (0, 0)),
            pl.BlockSpec((1, C), lambda i: (0, 0)),
            pl.BlockSpec((C, C), lambda i: (0, 0)),
            pl.BlockSpec((1, C), lambda i: (0, 0)),
            pl.BlockSpec((P, C), lambda i: (0, 0)),
        ],
        out_specs=[
            pl.BlockSpec((bm, C), lambda i: (i, 0)),
            pl.BlockSpec((bm, HW), lambda i: (i, 0)),
            pl.BlockSpec((bm, HW), lambda i: (i, 0)),
        ],
        out_shape=[
            jax.ShapeDtypeStruct((n, C), jnp.float32),
            jax.ShapeDtypeStruct((n, HW), jnp.float32),
            jax.ShapeDtypeStruct((n, HW), jnp.float32),
        ],
    )(x, pos, wk, bk.reshape(1, C), wv, bv.reshape(1, C), wp)


def _exp_shift(logits):
    e = logits.shape[0]
    lr = logits.reshape(e // 128, 128)

    def f(l_ref, e_ref):
        x = l_ref[...]
        m = jnp.max(x)
        e_ref[...] = jnp.exp(x - m)

    out = pl.pallas_call(
        f, out_shape=jax.ShapeDtypeStruct(lr.shape, jnp.float32))(lr)
    return out.reshape(e)


def _post_attn(acc_a, acc_b, den_a, den_b, pos, x, wp, bp):
    n = x.shape[0]
    grid, bm = _row_grid(n)

    def f(aa_ref, ab_ref, da_ref, db_ref, pos_ref, x_ref, wp_ref, bp_ref,
          r_ref, s1_ref, s2_ref):
        den = da_ref[...][:, :1] + db_ref[...][:, :1]
        invd = 1.0 / (den + 1e-16)
        s = den * invd
        acc = jnp.concatenate([aa_ref[...], ab_ref[...]], axis=1)
        pp = jnp.dot(pos_ref[...], wp_ref[...],
                     preferred_element_type=jnp.float32)
        r = x_ref[...] + acc * invd + s * (bp_ref[...] - pp)
        r_ref[...] = r

        @pl.when(pl.program_id(0) == 0)
        def _():
            s1_ref[...] = jnp.zeros_like(s1_ref)
            s2_ref[...] = jnp.zeros_like(s2_ref)

        s1_ref[...] += jnp.sum(r, axis=0, keepdims=True)
        s2_ref[...] += jnp.sum(r * r, axis=0, keepdims=True)

    return pl.pallas_call(
        f,
        grid=(grid,),
        in_specs=[
            pl.BlockSpec((bm, HW), lambda i: (i, 0)),
            pl.BlockSpec((bm, HW), lambda i: (i, 0)),
            pl.BlockSpec((bm, HW), lambda i: (i, 0)),
            pl.BlockSpec((bm, HW), lambda i: (i, 0)),
            pl.BlockSpec((bm, P), lambda i: (i, 0)),
            pl.BlockSpec((bm, C), lambda i: (i, 0)),
            pl.BlockSpec((P, C), lambda i: (0, 0)),
            pl.BlockSpec((1, C), lambda i: (0, 0)),
        ],
        out_specs=[
            pl.BlockSpec((bm, C), lambda i: (i, 0)),
            pl.BlockSpec((1, C), lambda i: (0, 0)),
            pl.BlockSpec((1, C), lambda i: (0, 0)),
        ],
        out_shape=[
            jax.ShapeDtypeStruct((n, C), jnp.float32),
            jax.ShapeDtypeStruct((1, C), jnp.float32),
            jax.ShapeDtypeStruct((1, C), jnp.float32),
        ],
    )(acc_a, acc_b, den_a, den_b, pos, x, wp, bp.reshape(1, C))


def _bn_apply(r, s1, s2, g, b):
    n = r.shape[0]
    grid, bm = _row_grid(n)

    def f(r_ref, s1_ref, s2_ref, g_ref, b_ref, y_ref):
        mu = s1_ref[...] * (1.0 / n)
        var = s2_ref[...] * (1.0 / n) - mu * mu
        y_ref[...] = (g_ref[...] * (r_ref[...] - mu)
                      * lax.rsqrt(var + 1e-5) + b_ref[...])

    return pl.pallas_call(
        f,
        grid=(grid,),
        in_specs=[
            pl.BlockSpec((bm, C), lambda i: (i, 0)),
            pl.BlockSpec((1, C), lambda i: (0, 0)),
            pl.BlockSpec((1, C), lambda i: (0, 0)),
            pl.BlockSpec((1, C), lambda i: (0, 0)),
            pl.BlockSpec((1, C), lambda i: (0, 0)),
        ],
        out_specs=pl.BlockSpec((bm, C), lambda i: (i, 0)),
        out_shape=jax.ShapeDtypeStruct((n, C), jnp.float32),
    )(r, s1, s2, g.reshape(1, C), b.reshape(1, C))


def _mlp1(x, w1, b1):
    n = x.shape[0]
    grid, bm = _row_grid(n)

    def f(x_ref, w_ref, b_ref, h_ref, s1_ref, s2_ref):
        h = jnp.dot(x_ref[...], w_ref[...], preferred_element_type=jnp.float32)
        h = h + b_ref[...]
        h_ref[...] = h

        @pl.when(pl.program_id(0) == 0)
        def _():
            s1_ref[...] = jnp.zeros_like(s1_ref)
            s2_ref[...] = jnp.zeros_like(s2_ref)

        s1_ref[...] += jnp.sum(h, axis=0, keepdims=True)
        s2_ref[...] += jnp.sum(h * h, axis=0, keepdims=True)

    return pl.pallas_call(
        f,
        grid=(grid,),
        in_specs=[
            pl.BlockSpec((bm, C), lambda i: (i, 0)),
            pl.BlockSpec((C, C), lambda i: (0, 0)),
            pl.BlockSpec((1, C), lambda i: (0, 0)),
        ],
        out_specs=[
            pl.BlockSpec((bm, C), lambda i: (i, 0)),
            pl.BlockSpec((1, C), lambda i: (0, 0)),
            pl.BlockSpec((1, C), lambda i: (0, 0)),
        ],
        out_shape=[
            jax.ShapeDtypeStruct((n, C), jnp.float32),
            jax.ShapeDtypeStruct((1, C), jnp.float32),
            jax.ShapeDtypeStruct((1, C), jnp.float32),
        ],
    )(x, w1, b1.reshape(1, C))


def _mlp2(h0, s1h, s2h, g1, be1, w2a, b2a, w2b, b2b, cres, pos):
    n = h0.shape[0]
    grid, bm = _row_grid(n)

    def f(h0_ref, s1_ref, s2_ref, g_ref, be_ref, w2a_ref, b2a_ref, w2b_ref,
          b2b_ref, c_ref, pos_ref, r_ref, s1r_ref, s2r_ref, posn_ref):
        mu = s1_ref[...] * (1.0 / n)
        var = s2_ref[...] * (1.0 / n) - mu * mu
        h = g_ref[...] * (h0_ref[...] - mu) * lax.rsqrt(var + 1e-5) + be_ref[...]
        h = jnp.where(h >= 0, h, 0.01 * h)
        out = jnp.dot(h, w2a_ref[...], preferred_element_type=jnp.float32)
        r = c_ref[...] + out + b2a_ref[...]
        r_ref[...] = r
        outp = jnp.dot(h, w2b_ref[...], preferred_element_type=jnp.float32)
        outp = outp + b2b_ref[...]
        posn_ref[...] = pos_ref[...] + outp[:, :P] * RADIUS

        @pl.when(pl.program_id(0) == 0)
        def _():
            s1r_ref[...] = jnp.zeros_like(s1r_ref)
            s2r_ref[...] = jnp.zeros_like(s2r_ref)

        s1r_ref[...] += jnp.sum(r, axis=0, keepdims=True)
        s2r_ref[...] += jnp.sum(r * r, axis=0, keepdims=True)

    return pl.pallas_call(
        f,
        grid=(grid,),
        in_specs=[
            pl.BlockSpec((bm, C), lambda i: (i, 0)),
            pl.BlockSpec((1, C), lambda i: (0, 0)),
            pl.BlockSpec((1, C), lambda i: (0, 0)),
            pl.BlockSpec((1, C), lambda i: (0, 0)),
            pl.BlockSpec((1, C), lambda i: (0, 0)),
            pl.BlockSpec((C, C), lambda i: (0, 0)),
            pl.BlockSpec((1, C), lambda i: (0, 0)),
            pl.BlockSpec((C, 128), lambda i: (0, 0)),
            pl.BlockSpec((1, 128), lambda i: (0, 0)),
            pl.BlockSpec((bm, C), lambda i: (i, 0)),
            pl.BlockSpec((bm, P), lambda i: (i, 0)),
        ],
        out_specs=[
            pl.BlockSpec((bm, C), lambda i: (i, 0)),
            pl.BlockSpec((1, C), lambda i: (0, 0)),
            pl.BlockSpec((1, C), lambda i: (0, 0)),
            pl.BlockSpec((bm, P), lambda i: (i, 0)),
        ],
        out_shape=[
            jax.ShapeDtypeStruct((n, C), jnp.float32),
            jax.ShapeDtypeStruct((1, C), jnp.float32),
            jax.ShapeDtypeStruct((1, C), jnp.float32),
            jax.ShapeDtypeStruct((n, P), jnp.float32),
        ],
    )(h0, s1h, s2h, g1.reshape(1, C), be1.reshape(1, C), w2a,
      b2a.reshape(1, C), w2b, b2b.reshape(1, 128), cres, pos)


def _sc_mesh():
    return plsc.VectorSubcoreMesh(core_axis_name="c", subcore_axis_name="s",
                                  num_cores=NC, num_subcores=NS)

_SC_PARAMS = pltpu.CompilerParams(needs_layout_passes=False)


def _lane_bcast(vec, l):
    lane = jnp.sum(jnp.where(lax.iota(jnp.int32, 16) == l, vec, 0.0))
    return jnp.broadcast_to(lane, (16,))


def _sc_logits(qx, kx, src, dst):
    e_total = src.shape[0]
    ch = 80
    assert e_total % ch == 0
    nck = e_total // ch
    nw = NC * NS
    per = (nck + nw - 1) // nw

    def body(qx_ref, kx_ref, src_ref, dst_ref, out_ref,
             idx_s, idx_d, qrows, krows, lg, sem):
        w = lax.axis_index("s") * NC + lax.axis_index("c")
        iota = lax.iota(jnp.int32, 16)

        def chunk(q_, carry):
            k = w + q_ * nw

            @pl.when(k < nck)
            def _():
                base = k * ch
                pltpu.sync_copy(src_ref.at[pl.ds(base, ch)], idx_s)
                pltpu.sync_copy(dst_ref.at[pl.ds(base, ch)], idx_d)
                pltpu.async_copy(kx_ref.at[idx_s], krows, sem).wait()
                pltpu.async_copy(qx_ref.at[idx_d], qrows, sem).wait()
                for g in range(ch // 16):
                    lgv = jnp.zeros((16,), jnp.float32)
                    for l in range(16):
                        j = g * 16 + l
                        acc = qrows[j, pl.ds(0, 16)] * krows[j, pl.ds(0, 16)]
                        for t in range(1, C // 16):
                            acc = acc + (qrows[j, pl.ds(t * 16, 16)]
                                         * krows[j, pl.ds(t * 16, 16)])
                        lgv = jnp.where(iota == l, jnp.sum(acc), lgv)
                    lg[pl.ds(g * 16, 16)] = lgv
                pltpu.sync_copy(lg, out_ref.at[pl.ds(base, ch)])

            return carry

        lax.fori_loop(0, per, chunk, 0)

    return pl.kernel(
        body,
        out_type=jax.ShapeDtypeStruct((e_total,), jnp.float32),
        mesh=_sc_mesh(),
        compiler_params=_SC_PARAMS,
        scratch_types=[
            pltpu.VMEM((ch,), jnp.int32),
            pltpu.VMEM((ch,), jnp.int32),
            pltpu.VMEM((ch, C), jnp.float32),
            pltpu.VMEM((ch, C), jnp.float32),
            pltpu.VMEM((ch,), jnp.float32),
            pltpu.SemaphoreType.DMA,
        ],
    )(qx, kx, src, dst)


def _sc_aggregate(ua, ub, ev, src, dst, zeros_nd):
    e_total = src.shape[0]
    nd = zeros_nd.shape[0]
    per_t = e_total // NS
    ch = 80
    assert per_t % ch == 0
    nch = per_t // ch

    def body(ua_ref, ub_ref, e_ref, src_ref, dst_ref, z_ref,
             acca_ref, accb_ref, idx_s, idx_d, e_v, rows, acc_sp, sem):
        cid = lax.axis_index("c")
        sid = lax.axis_index("s")

        @pl.when(sid == 0)
        def _():
            pltpu.sync_copy(z_ref, acc_sp)

        plsc.subcore_barrier()

        def run(tbl_ref):
            def chunk(i, carry):
                base = sid * per_t + i * ch
                pltpu.sync_copy(src_ref.at[pl.ds(base, ch)], idx_s)
                pltpu.sync_copy(dst_ref.at[pl.ds(base, ch)], idx_d)
                pltpu.sync_copy(e_ref.at[pl.ds(base, ch)], e_v)
                pltpu.async_copy(tbl_ref.at[idx_s], rows, sem).wait()
                for g in range(ch // 16):
                    ev_vec = e_v[pl.ds(g * 16, 16)]
                    for l in range(16):
                        j = g * 16 + l
                        sp = _lane_bcast(ev_vec, l)
                        for t in range(HW // 16):
                            rows[j, pl.ds(t * 16, 16)] = (
                                rows[j, pl.ds(t * 16, 16)] * sp)
                pltpu.sync_copy(rows, acc_sp.at[idx_d], add=True)
                return carry

            lax.fori_loop(0, nch, chunk, 0)

        @pl.when(cid == 0)
        def _():
            run(ua_ref)

        @pl.when(cid == 1)
        def _():
            run(ub_ref)

        plsc.subcore_barrier()

        @pl.when(jnp.logical_and(sid == 0, cid == 0))
        def _():
            pltpu.sync_copy(acc_sp, acca_ref)

        @pl.when(jnp.logical_and(sid == 0, cid == 1))
        def _():
            pltpu.sync_copy(acc_sp, accb_ref)

    return pl.kernel(
        body,
        out_type=(jax.ShapeDtypeStruct((nd, HW), jnp.float32),
                  jax.ShapeDtypeStruct((nd, HW), jnp.float32)),
        mesh=_sc_mesh(),
        compiler_params=_SC_PARAMS,
        scratch_types=[
            pltpu.VMEM((ch,), jnp.int32),
            pltpu.VMEM((ch,), jnp.int32),
            pltpu.VMEM((ch,), jnp.float32),
            pltpu.VMEM((ch, HW), jnp.float32),
            pltpu.VMEM_SHARED((nd, HW), jnp.float32),
            pltpu.SemaphoreType.DMA,
        ],
    )(ua, ub, ev, src, dst, zeros_nd)


def _sc_den(ev, dst, zeros_nd):
    e_total = dst.shape[0]
    nd = zeros_nd.shape[0]
    ch = 80
    assert e_total % ch == 0
    nck = e_total // ch
    nw = NC * NS
    per = (nck + nw - 1) // nw

    def body(e_ref, dst_ref, z_ref, dena_ref, denb_ref,
             idx_d, e_v, rows, acc_sp, sem):
        cid = lax.axis_index("c")
        sid = lax.axis_index("s")
        w = sid * NC + cid
        one0 = (lax.iota(jnp.int32, 16) == 0).astype(jnp.float32)
        zv = jnp.zeros((16,), jnp.float32)

        @pl.when(sid == 0)
        def _():
            pltpu.sync_copy(z_ref, acc_sp)

        for j in range(ch):
            for t in range(HW // 16):
                rows[j, pl.ds(t * 16, 16)] = zv

        plsc.subcore_barrier()

        def chunk(q_, carry):
            k = w + q_ * nw

            @pl.when(k < nck)
            def _():
                base = k * ch
                pltpu.sync_copy(dst_ref.at[pl.ds(base, ch)], idx_d)
                pltpu.sync_copy(e_ref.at[pl.ds(base, ch)], e_v)
                for g in range(ch // 16):
                    ev_vec = e_v[pl.ds(g * 16, 16)]
                    for l in range(16):
                        j = g * 16 + l
                        rows[j, pl.ds(0, 16)] = _lane_bcast(ev_vec, l) * one0
                pltpu.sync_copy(rows, acc_sp.at[idx_d], add=True)

            return carry

        lax.fori_loop(0, per, chunk, 0)
        plsc.subcore_barrier()

        @pl.when(jnp.logical_and(sid == 0, cid == 0))
        def _():
            pltpu.sync_copy(acc_sp, dena_ref)

        @pl.when(jnp.logical_and(sid == 0, cid == 1))
        def _():
            pltpu.sync_copy(acc_sp, denb_ref)

    return pl.kernel(
        body,
        out_type=(jax.ShapeDtypeStruct((nd, HW), jnp.float32),
                  jax.ShapeDtypeStruct((nd, HW), jnp.float32)),
        mesh=_sc_mesh(),
        compiler_params=_SC_PARAMS,
        scratch_types=[
            pltpu.VMEM((ch,), jnp.int32),
            pltpu.VMEM((ch,), jnp.float32),
            pltpu.VMEM((ch, HW), jnp.float32),
            pltpu.VMEM_SHARED((nd, HW), jnp.float32),
            pltpu.SemaphoreType.DMA,
        ],
    )(ev, dst, zeros_nd)


def _attention(x_src, x_dst, pos_src, src, dst, p, zeros_nd):
    qx = _prep_dst(x_dst, p["Wq"], p["bq"])
    kx, ua, ub = _prep_src(x_src, pos_src, p["Wk"], p["bk"], p["Wv"],
                           p["bv"], p["Wp"])
    logits = _sc_logits(qx, kx, src, dst)
    ev = _exp_shift(logits)
    acc_a, acc_b = _sc_aggregate(ua, ub, ev, src, dst, zeros_nd)
    den_a, den_b = _sc_den(ev, dst, zeros_nd)
    return acc_a, acc_b, den_a, den_b


def kernel(encoding, encoding_pos, object, object_pos, edge_index_object,
           edge_index_cross, params):
    obj = object
    pos = object_pos
    n_obj = obj.shape[0]
    so, do_ = edge_index_object[0], edge_index_object[1]
    sc_, dc_ = edge_index_cross[0], edge_index_cross[1]
    zeros_nd = jnp.zeros((n_obj, HW), jnp.float32)

    for lp in params:
        pa = lp["self"]
        aa, ab, da, db = _attention(obj, obj, pos, so, do_, pa, zeros_nd)
        r, s1, s2 = _post_attn(aa, ab, da, db, pos, obj, pa["Wp"], pa["bp"])
        s_ = _bn_apply(r, s1, s2, lp["ns_g"], lp["ns_b"])

        pc = lp["cross"]
        aa, ab, da, db = _attention(encoding, s_, encoding_pos, sc_, dc_,
                                    pc, zeros_nd)
        r, s1, s2 = _post_attn(aa, ab, da, db, pos, s_, pc["Wp"], pc["bp"])
        c_ = _bn_apply(r, s1, s2, lp["nc_g"], lp["nc_b"])

        mp = lp["mlp"]
        w2a = mp["W2"][:, :C]
        w2b = jnp.pad(mp["W2"][:, C:], ((0, 0), (0, 128 - P)))
        b2a = mp["b2"][:C]
        b2b = jnp.pad(mp["b2"][C:], (0, 128 - P))
        h0, s1h, s2h = _mlp1(c_, mp["W1"], mp["b1"])
        r3, s1r, s2r, pos = _mlp2(h0, s1h, s2h, mp["g1"], mp["be1"],
                                  w2a, b2a, w2b, b2b, c_, pos)
        obj = _bn_apply(r3, s1r, s2r, lp["nm_g"], lp["nm_b"])

    return obj, pos
